# baseline (speedup 1.0000x reference)
import functools

import numpy as np
import jax
import jax.numpy as jnp
from jax import lax
from jax.experimental import pallas as pl
from jax.experimental.pallas import tpu as pltpu
from jax.experimental.pallas import tpu_sc as plsc

L = 200
N_CAT = 26
N_CONT = 13
WC = L * N_CAT
WF = L * N_CONT
VOCAB = 100000
S_SC = 512
RBLK = 64
_ROT_A = (13, 15, 26, 6)
_ROT_B = (17, 29, 16, 24)


def _tf_host(k1, k2, x0, x1):
    M = 0xFFFFFFFF

    def rotl(v, r):
        return ((v << r) | (v >> (32 - r))) & M

    ks = (k1, k2, (k1 ^ k2 ^ 0x1BD11BDA) & M)
    x0 = (x0 + ks[0]) & M
    x1 = (x1 + ks[1]) & M
    for i in range(5):
        for r in (_ROT_A if i % 2 == 0 else _ROT_B):
            x0 = (x0 + x1) & M
            x1 = rotl(x1, r)
            x1 ^= x0
        x0 = (x0 + ks[(i + 1) % 3]) & M
        x1 = (x1 + ks[(i + 2) % 3] + i + 1) & M
    return x0, x1


_SUBS = [_tf_host(0, 42, 0, i) for i in range(5)]
_K_KEEP, _K_SUB, _K_MASK, _K_GAU, _K_SHUF = _SUBS
_K_RAND = _tf_host(_K_SUB[0], _K_SUB[1], 0, 1)


def _i32c(v):
    return jnp.int32(np.uint32(v).astype(np.int32))


def _rotl_v(x, r):
    return lax.shift_left(x, jnp.int32(r)) | lax.shift_right_logical(
        x, jnp.int32(32 - r))


def _tf_bits(key, x1):
    k1, k2 = key
    ks = (_i32c(k1), _i32c(k2), _i32c((k1 ^ k2 ^ 0x1BD11BDA) & 0xFFFFFFFF))
    x0 = jnp.full(x1.shape, ks[0], dtype=jnp.int32)
    x1 = x1 + ks[1]
    for i in range(5):
        for r in (_ROT_A if i % 2 == 0 else _ROT_B):
            x0 = x0 + x1
            x1 = _rotl_v(x1, r)
            x1 = x1 ^ x0
        x0 = x0 + ks[(i + 1) % 3]
        x1 = x1 + (ks[(i + 2) % 3] + jnp.int32(i + 1))
    return x0 ^ x1


def _bits_to_unit_f32(bits):
    m = lax.shift_right_logical(bits, jnp.int32(9)) | _i32c(0x3F800000)
    return lax.bitcast_convert_type(m, jnp.float32) - jnp.float32(1.0)


def _umod_vocab(bits):
    xf = bits.astype(jnp.float32) + jnp.where(
        bits < 0, jnp.float32(4294967296.0), jnp.float32(0.0))
    q = (xf * jnp.float32(1.0 / VOCAB)).astype(jnp.int32)
    r = bits - q * jnp.int32(VOCAB)
    r = r + jnp.where(r < 0, jnp.int32(VOCAB), jnp.int32(0))
    r = r - jnp.where(r >= VOCAB, jnp.int32(VOCAB), jnp.int32(0))
    return r


def _erfinv_f32(u):
    w = -jnp.log1p(-u * u)
    lt = w < jnp.float32(5.0)
    w1 = w - jnp.float32(2.5)
    w2 = jnp.sqrt(jnp.maximum(w, jnp.float32(5.0))) - jnp.float32(3.0)
    c1 = (2.81022636e-08, 3.43273939e-07, -3.5233877e-06, -4.39150654e-06,
          0.00021858087, -0.00125372503, -0.00417768164, 0.246640727,
          1.50140941)
    c2 = (-0.000200214257, 0.000100950558, 0.00134934322, -0.00367342844,
          0.00573950773, -0.0076224613, 0.00943887047, 1.00167406,
          2.83297682)
    p1 = jnp.float32(c1[0])
    for c in c1[1:]:
        p1 = p1 * w1 + jnp.float32(c)
    p2 = jnp.float32(c2[0])
    for c in c2[1:]:
        p2 = p2 * w2 + jnp.float32(c)
    return jnp.where(lt, p1, p2) * u


_M_GT = 0x0CCCCC
_M_LT = 0x0CCCCD

_LO_NORM = float(np.nextafter(np.float32(-1.0), np.float32(0.0)))
_SQRT2 = 1.4142135381698608


def _shuffle_masks(keyv, li):
    tn = (jnp.roll(keyv, -1, axis=1) < keyv) & (li < jnp.int32(L - 1))
    tp = (keyv < jnp.roll(keyv, 1, axis=1)) & (li > jnp.int32(0))
    return tn, tp


def _cont_block(cont, idx_f):
    gmask_m = lax.shift_right_logical(_tf_bits(_K_MASK, idx_f), jnp.int32(9))
    gmask = (gmask_m < jnp.int32(_M_LT)).astype(jnp.float32)
    gu = _bits_to_unit_f32(_tf_bits(_K_GAU, idx_f))
    un = jnp.maximum(jnp.float32(_LO_NORM),
                     gu * jnp.float32(2.0) + jnp.float32(_LO_NORM))
    z = jnp.float32(_SQRT2) * _erfinv_f32(un)
    return cont + z * gmask


def _build_e(rep, width, ref):
    lv = lax.broadcasted_iota(jnp.int32, (L, width), 0)
    jv = lax.broadcasted_iota(jnp.int32, (L, width), 1)
    t = jv - jnp.int32(rep) * lv
    ref[...] = ((t >= 0) & (t < rep)).astype(jnp.bfloat16)


def _build_iota(width, rblk, ref):
    iv = lax.broadcasted_iota(jnp.int32, (rblk, width), 0)
    jv = lax.broadcasted_iota(jnp.int32, (rblk, width), 1)
    ref[...] = iv * jnp.int32(width) + jv


_DN = (((1,), (0,)), ((), ()))


def _apply_shuffle(tn_bf, tp_bf, rep, data, out_ref, e_ref):
    tn_w = lax.dot_general(tn_bf, e_ref[...], _DN,
                           preferred_element_type=jnp.float32)
    tp_w = lax.dot_general(tp_bf, e_ref[...], _DN,
                           preferred_element_type=jnp.float32)
    half = jnp.float32(0.5)
    shifted = jnp.where(tn_w > half, jnp.roll(data, -rep, axis=1),
                        jnp.roll(data, rep, axis=1))
    out_ref[...] = jnp.where((tn_w + tp_w) > half, shifted, data)


def _tc_main_kernel(cat_ref, cont_ref, cat_out_ref, cont_out_ref, e26_ref,
                    e13_ref, ic_ref, if_ref, il_ref):
    rblk = cat_ref.shape[0]
    row0 = pl.program_id(0) * rblk

    @pl.when(pl.program_id(0) == 0)
    def _():
        _build_e(N_CAT, WC, e26_ref)
        _build_e(N_CONT, WF, e13_ref)
        _build_iota(WC, rblk, ic_ref)
        _build_iota(WF, rblk, if_ref)
        _build_iota(L, rblk, il_ref)

    idx_c = ic_ref[...] + row0 * jnp.int32(WC)
    keep_m = lax.shift_right_logical(_tf_bits(_K_KEEP, idx_c), jnp.int32(9))
    rand_tok = _umod_vocab(_tf_bits(_K_RAND, idx_c))
    cat = jnp.where(keep_m > jnp.int32(_M_GT), cat_ref[...], rand_tok)

    cont = _cont_block(cont_ref[...], if_ref[...] + row0 * jnp.int32(WF))

    li = lax.broadcasted_iota(jnp.int32, (rblk, L), 1)
    inc_u = _bits_to_unit_f32(_tf_bits(_K_SHUF, il_ref[...] +
                                       row0 * jnp.int32(L)))
    keyv = li.astype(jnp.float32) + jnp.float32(2.0) * inc_u
    tn, tp = _shuffle_masks(keyv, li)
    tn_bf = tn.astype(jnp.bfloat16)
    tp_bf = tp.astype(jnp.bfloat16)
    _apply_shuffle(tn_bf, tp_bf, N_CAT, cat, cat_out_ref, e26_ref)
    _apply_shuffle(tn_bf, tp_bf, N_CONT, cont, cont_out_ref, e13_ref)


def _tc_cont_kernel(row_offset, cont_ref, cont_out_ref, e13_ref, if_ref,
                    il_ref):
    rblk = cont_ref.shape[0]
    row0 = jnp.int32(row_offset) + pl.program_id(0) * rblk

    @pl.when(pl.program_id(0) == 0)
    def _():
        _build_e(N_CONT, WF, e13_ref)
        _build_iota(WF, rblk, if_ref)
        _build_iota(L, rblk, il_ref)

    cont = _cont_block(cont_ref[...], if_ref[...] + row0 * jnp.int32(WF))
    li = lax.broadcasted_iota(jnp.int32, (rblk, L), 1)
    inc_u = _bits_to_unit_f32(_tf_bits(_K_SHUF, il_ref[...] +
                                       row0 * jnp.int32(L)))
    keyv = li.astype(jnp.float32) + jnp.float32(2.0) * inc_u
    tn, tp = _shuffle_masks(keyv, li)
    _apply_shuffle(tn.astype(jnp.bfloat16), tp.astype(jnp.bfloat16), N_CONT,
                   cont, cont_out_ref, e13_ref)


_PAD = 32


def _sc_cat_call(cat_flat, row_offset, s_rows):
    info = plsc.get_sparse_core_info()
    nc, ns = info.num_cores, info.num_subcores
    nw = nc * ns
    rpw = s_rows // nw
    n_chunks = WC // 16
    mesh = plsc.VectorSubcoreMesh(core_axis_name="c", subcore_axis_name="s")

    @functools.partial(
        pl.kernel, mesh=mesh,
        out_type=jax.ShapeDtypeStruct((s_rows * WC,), jnp.int32),
        scratch_types=[
            pltpu.VMEM((WC + 2 * _PAD,), jnp.int32),
            pltpu.VMEM((WC,), jnp.int32),
            pltpu.VMEM((224,), jnp.float32),
        ])
    def k(cat_hbm, out_hbm, noised, outbuf, keybuf):
        wid = lax.axis_index("s") * nc + lax.axis_index("c")

        def row_body(r, carry):
            row_l = wid * rpw + r
            row_g = jnp.int32(row_offset) + row_l
            pltpu.sync_copy(cat_hbm.at[pl.ds(row_g * WC, WC)],
                            noised.at[pl.ds(_PAD, WC)])

            def key_body(c, _):
                lv = c * 16 + lax.iota(jnp.int32, 16)
                u = _bits_to_unit_f32(_tf_bits(_K_SHUF, row_g * jnp.int32(L)
                                               + lv))
                kv = lv.astype(jnp.float32) + jnp.float32(2.0) * u
                keybuf[pl.ds(c * 16, 16)] = jnp.where(
                    lv < jnp.int32(L), kv, jnp.float32(1e9))
                return 0

            lax.fori_loop(0, 14, key_body, 0, unroll=False)

            def noise_body(c, _):
                j = c * 16 + lax.iota(jnp.int32, 16)
                idx = row_g * jnp.int32(WC) + j
                keep_m = lax.shift_right_logical(_tf_bits(_K_KEEP, idx),
                                                 jnp.int32(9))
                rand = _umod_vocab(_tf_bits(_K_RAND, idx))
                v = noised[pl.ds(c * 16 + _PAD, 16)]
                noised[pl.ds(c * 16 + _PAD, 16)] = jnp.where(
                    keep_m > jnp.int32(_M_GT), v, rand)
                return 0

            lax.fori_loop(0, n_chunks, noise_body, 0, unroll=False)

            def shuf_body(c, _):
                j = c * 16 + lax.iota(jnp.int32, 16)
                l0 = lax.div(c * 16, jnp.int32(N_CAT))
                l = ((j.astype(jnp.float32) + jnp.float32(0.5)) *
                     jnp.float32(1.0 / N_CAT)).astype(jnp.int32)
                in_l0 = l == l0
                v_c = keybuf[pl.ds(l0, 16)]
                v_m = keybuf[pl.ds(jnp.maximum(l0 - jnp.int32(1),
                                               jnp.int32(0)), 16)]
                k_m1 = v_m[0]
                k_0 = v_c[0]
                k_1 = v_c[1]
                k_2 = v_c[2]
                kl = jnp.where(in_l0, k_0, k_1)
                kn = jnp.where(in_l0, k_1, k_2)
                kp = jnp.where(in_l0, k_m1, k_0)
                tn = kn < kl
                tp = kl < kp
                v0 = noised[pl.ds(c * 16 + _PAD, 16)]
                vn = noised[pl.ds(c * 16 + _PAD + N_CAT, 16)]
                vp = noised[pl.ds(c * 16 + _PAD - N_CAT, 16)]
                outbuf[pl.ds(c * 16, 16)] = jnp.where(
                    tn, vn, jnp.where(tp, vp, v0))
                return 0

            lax.fori_loop(0, n_chunks, shuf_body, 0, unroll=False)
            pltpu.sync_copy(outbuf, out_hbm.at[pl.ds(row_l * WC, WC)])
            return carry

        lax.fori_loop(0, rpw, row_body, 0, unroll=False)

    return k(cat_flat).reshape(s_rows, WC)


def _tc_calls(cat2, cont2, head_rows):
    grid_h = head_rows // RBLK
    cat_o, cont_o = pl.pallas_call(
        _tc_main_kernel,
        grid=(grid_h,),
        in_specs=[
            pl.BlockSpec((RBLK, WC), lambda i: (i, 0)),
            pl.BlockSpec((RBLK, WF), lambda i: (i, 0)),
        ],
        out_specs=[
            pl.BlockSpec((RBLK, WC), lambda i: (i, 0)),
            pl.BlockSpec((RBLK, WF), lambda i: (i, 0)),
        ],
        out_shape=[
            jax.ShapeDtypeStruct((head_rows, WC), jnp.int32),
            jax.ShapeDtypeStruct((head_rows, WF), jnp.float32),
        ],
        scratch_shapes=[
            pltpu.VMEM((L, WC), jnp.bfloat16),
            pltpu.VMEM((L, WF), jnp.bfloat16),
            pltpu.VMEM((RBLK, WC), jnp.int32),
            pltpu.VMEM((RBLK, WF), jnp.int32),
            pltpu.VMEM((RBLK, L), jnp.int32),
        ],
        compiler_params=pltpu.CompilerParams(
            dimension_semantics=("arbitrary",)),
    )(cat2, cont2)

    s_rows = cat2.shape[0] - head_rows
    tail0 = head_rows // RBLK
    cont_t = pl.pallas_call(
        functools.partial(_tc_cont_kernel, head_rows),
        grid=(s_rows // RBLK,),
        in_specs=[pl.BlockSpec((RBLK, WF), lambda i: (tail0 + i, 0))],
        out_specs=pl.BlockSpec((RBLK, WF), lambda i: (i, 0)),
        out_shape=jax.ShapeDtypeStruct((s_rows, WF), jnp.float32),
        scratch_shapes=[
            pltpu.VMEM((L, WF), jnp.bfloat16),
            pltpu.VMEM((RBLK, WF), jnp.int32),
            pltpu.VMEM((RBLK, L), jnp.int32),
        ],
        compiler_params=pltpu.CompilerParams(
            dimension_semantics=("arbitrary",)),
    )(cont2)
    return cat_o, cont_o, cont_t


def kernel(seq_cat_data, seq_cont_data):
    b = seq_cat_data.shape[0]
    head = b - S_SC
    cat2 = seq_cat_data.reshape(b, WC)
    cont2 = seq_cont_data.reshape(b, WF)
    cat_h, cont_h, cont_t = _tc_calls(cat2, cont2, head)
    cat_t = _sc_cat_call(cat2.reshape(-1), head, S_SC)
    cat_o = jnp.concatenate([cat_h, cat_t], axis=0)
    cont_o = jnp.concatenate([cont_h, cont_t], axis=0)
    return (cat_o.reshape(b, L, N_CAT), cont_o.reshape(b, L, N_CONT))


# speedup vs baseline: 1.2801x; 1.2801x over previous
import numpy as np
import jax
import jax.numpy as jnp
from jax import lax
from jax.experimental import pallas as pl
from jax.experimental.pallas import tpu as pltpu

L = 200
N_CAT = 26
N_CONT = 13
VOCAB = 100000
_ROT_A = (13, 15, 26, 6)
_ROT_B = (17, 29, 16, 24)


def _tf_host(k1, k2, x0, x1):
    M = 0xFFFFFFFF

    def rotl(v, r):
        return ((v << r) | (v >> (32 - r))) & M

    ks = (k1, k2, (k1 ^ k2 ^ 0x1BD11BDA) & M)
    x0 = (x0 + ks[0]) & M
    x1 = (x1 + ks[1]) & M
    for i in range(5):
        for r in (_ROT_A if i % 2 == 0 else _ROT_B):
            x0 = (x0 + x1) & M
            x1 = rotl(x1, r)
            x1 ^= x0
        x0 = (x0 + ks[(i + 1) % 3]) & M
        x1 = (x1 + ks[(i + 2) % 3] + i + 1) & M
    return x0, x1


_SUBS = [_tf_host(0, 42, 0, i) for i in range(5)]
_K_KEEP, _K_SUB, _K_MASK, _K_GAU, _K_SHUF = _SUBS
_K_RAND = _tf_host(_K_SUB[0], _K_SUB[1], 0, 1)


def _i32c(v):
    return jnp.int32(np.uint32(v).astype(np.int32))


def _rotl_v(x, r):
    return lax.shift_left(x, jnp.int32(r)) | lax.shift_right_logical(
        x, jnp.int32(32 - r))


def _tf_bits(key, x1):
    k1, k2 = key
    ks = (_i32c(k1), _i32c(k2), _i32c((k1 ^ k2 ^ 0x1BD11BDA) & 0xFFFFFFFF))
    x0 = jnp.full(x1.shape, ks[0], dtype=jnp.int32)
    x1 = x1 + ks[1]
    for i in range(5):
        for r in (_ROT_A if i % 2 == 0 else _ROT_B):
            x0 = x0 + x1
            x1 = _rotl_v(x1, r)
            x1 = x1 ^ x0
        x0 = x0 + ks[(i + 1) % 3]
        x1 = x1 + (ks[(i + 2) % 3] + jnp.int32(i + 1))
    return x0 ^ x1


def _bits_to_unit_f32(bits):
    m = lax.shift_right_logical(bits, jnp.int32(9)) | _i32c(0x3F800000)
    return lax.bitcast_convert_type(m, jnp.float32) - jnp.float32(1.0)


def _umod_vocab(bits):
    xf = bits.astype(jnp.float32) + jnp.where(
        bits < 0, jnp.float32(4294967296.0), jnp.float32(0.0))
    q = (xf * jnp.float32(1.0 / VOCAB)).astype(jnp.int32)
    r = bits - q * jnp.int32(VOCAB)
    r = r + jnp.where(r < 0, jnp.int32(VOCAB), jnp.int32(0))
    r = r - jnp.where(r >= VOCAB, jnp.int32(VOCAB), jnp.int32(0))
    return r


def _erfinv_f32(u):
    w = -jnp.log1p(-u * u)
    lt = w < jnp.float32(5.0)
    w1 = w - jnp.float32(2.5)
    w2 = jnp.sqrt(jnp.maximum(w, jnp.float32(5.0))) - jnp.float32(3.0)
    c1 = (2.81022636e-08, 3.43273939e-07, -3.5233877e-06, -4.39150654e-06,
          0.00021858087, -0.00125372503, -0.00417768164, 0.246640727,
          1.50140941)
    c2 = (-0.000200214257, 0.000100950558, 0.00134934322, -0.00367342844,
          0.00573950773, -0.0076224613, 0.00943887047, 1.00167406,
          2.83297682)
    p1 = jnp.float32(c1[0])
    for c in c1[1:]:
        p1 = p1 * w1 + jnp.float32(c)
    p2 = jnp.float32(c2[0])
    for c in c2[1:]:
        p2 = p2 * w2 + jnp.float32(c)
    return jnp.where(lt, p1, p2) * u


_M_GT = 0x0CCCCC
_M_LT = 0x0CCCCD


def _noise_kernel(cat_ref, cont_ref, cat_out_ref, cont_out_ref, e26_ref,
                  e13_ref, ic_ref, if_ref, il_ref):
    rblk = cat_ref.shape[0]
    wc = L * N_CAT
    wf = L * N_CONT
    row0 = pl.program_id(0) * rblk

    @pl.when(pl.program_id(0) == 0)
    def _():
        for rep, width, ref in ((N_CAT, wc, e26_ref), (N_CONT, wf, e13_ref)):
            lv = lax.broadcasted_iota(jnp.int32, (L, width), 0)
            jv = lax.broadcasted_iota(jnp.int32, (L, width), 1)
            t = jv - jnp.int32(rep) * lv
            one = (t >= 0) & (t < rep)
            ref[...] = one.astype(jnp.bfloat16)
        for width, ref in ((wc, ic_ref), (wf, if_ref), (L, il_ref)):
            iv = lax.broadcasted_iota(jnp.int32, (rblk, width), 0)
            jv = lax.broadcasted_iota(jnp.int32, (rblk, width), 1)
            ref[...] = iv * jnp.int32(width) + jv

    idx_c = ic_ref[...] + row0 * jnp.int32(wc)
    keep_m = lax.shift_right_logical(_tf_bits(_K_KEEP, idx_c), jnp.int32(9))
    rand_tok = _umod_vocab(_tf_bits(_K_RAND, idx_c))
    cat = jnp.where(keep_m > jnp.int32(_M_GT), cat_ref[...], rand_tok)

    idx_f = if_ref[...] + row0 * jnp.int32(wf)
    gmask_m = lax.shift_right_logical(_tf_bits(_K_MASK, idx_f), jnp.int32(9))
    gmask = (gmask_m < jnp.int32(_M_LT)).astype(jnp.float32)
    gu = _bits_to_unit_f32(_tf_bits(_K_GAU, idx_f))
    lo = jnp.float32(np.nextafter(np.float32(-1.0), np.float32(0.0)))
    un = jnp.maximum(lo, gu * jnp.float32(2.0) + lo)
    z = jnp.float32(1.4142135381698608) * _erfinv_f32(un)
    cont = cont_ref[...] + z * gmask

    li = lax.broadcasted_iota(jnp.int32, (rblk, L), 1)
    inc_u = _bits_to_unit_f32(
        _tf_bits(_K_SHUF, il_ref[...] + row0 * jnp.int32(L)))
    keyv = li.astype(jnp.float32) + jnp.float32(2.0) * inc_u
    tn = (jnp.roll(keyv, -1, axis=1) < keyv) & (li < jnp.int32(L - 1))
    tp = (keyv < jnp.roll(keyv, 1, axis=1)) & (li > jnp.int32(0))
    tn_bf = tn.astype(jnp.bfloat16)
    tp_bf = tp.astype(jnp.bfloat16)
    dn = (((1,), (0,)), ((), ()))

    for rep, data, out_ref, e_ref in ((N_CAT, cat, cat_out_ref, e26_ref),
                                      (N_CONT, cont, cont_out_ref, e13_ref)):
        tn_w = lax.dot_general(tn_bf, e_ref[...], dn,
                               preferred_element_type=jnp.float32)
        tp_w = lax.dot_general(tp_bf, e_ref[...], dn,
                               preferred_element_type=jnp.float32)
        half = jnp.float32(0.5)
        shifted = jnp.where(tn_w > half, jnp.roll(data, -rep, axis=1),
                            jnp.roll(data, rep, axis=1))
        out_ref[...] = jnp.where((tn_w + tp_w) > half, shifted, data)


def kernel(seq_cat_data, seq_cont_data):
    b = seq_cat_data.shape[0]
    rblk = 64
    cat2 = seq_cat_data.reshape(b, L * N_CAT)
    cont2 = seq_cont_data.reshape(b, L * N_CONT)
    cat_o, cont_o = pl.pallas_call(
        _noise_kernel,
        grid=(b // rblk,),
        in_specs=[
            pl.BlockSpec((rblk, L * N_CAT), lambda i: (i, 0)),
            pl.BlockSpec((rblk, L * N_CONT), lambda i: (i, 0)),
        ],
        out_specs=[
            pl.BlockSpec((rblk, L * N_CAT), lambda i: (i, 0)),
            pl.BlockSpec((rblk, L * N_CONT), lambda i: (i, 0)),
        ],
        out_shape=[
            jax.ShapeDtypeStruct((b, L * N_CAT), seq_cat_data.dtype),
            jax.ShapeDtypeStruct((b, L * N_CONT), seq_cont_data.dtype),
        ],
        scratch_shapes=[
            pltpu.VMEM((L, L * N_CAT), jnp.bfloat16),
            pltpu.VMEM((L, L * N_CONT), jnp.bfloat16),
            pltpu.VMEM((rblk, L * N_CAT), jnp.int32),
            pltpu.VMEM((rblk, L * N_CONT), jnp.int32),
            pltpu.VMEM((rblk, L), jnp.int32),
        ],
        compiler_params=pltpu.CompilerParams(
            dimension_semantics=("arbitrary",)),
    )(cat2, cont2)
    return (cat_o.reshape(b, L, N_CAT), cont_o.reshape(b, L, N_CONT))
